# baseline (speedup 1.0000x reference)
import functools

import jax
import jax.numpy as jnp
from jax import lax
from jax.experimental import pallas as pl
from jax.experimental.pallas import tpu as pltpu
from jax.experimental.pallas import tpu_sc as plsc

_N = 100000
_E = 3200000
_G = 64

_NC = 2
_NSC = 16
_NW = _NC * _NSC

_IW = 128
_KJ = 2
_EP = 3276800
_R = _EP // _IW
_RW = _R // _NW
_NM = _RW // _KJ
_NB = 4
_SUP = _NB * _KJ
_NQ = _NM // _NB

_NP = 100352
_NROWS = _NP // _NSC
_NZ = _NROWS // _IW

_BT = 1000
_GRID = _N // _BT


@functools.cache
def _sc_mesh():
    return plsc.VectorSubcoreMesh(
        core_axis_name="c", subcore_axis_name="s",
        num_cores=_NC, num_subcores=_NSC)


def _fill_rows(buf, n, val):
    @pl.loop(0, n)
    def _(i):
        buf[i] = jnp.full((16,), val, jnp.float32)


def _zero_fill(acc, zbuf, base, semz):
    @pl.loop(0, _NZ)
    def _(k):
        pltpu.async_copy(zbuf, acc.at[pl.ds(base + k * _IW, _IW)], semz)

    @pl.loop(0, _NZ)
    def _(k):
        pltpu.make_async_copy(
            zbuf, acc.at[pl.ds(base, _IW)], semz).wait()


def _deg_body(dst_hbm, out_hbm, acc, idxd_v, ones_v, semi, semsc, semz):
    c = lax.axis_index("c")
    s = lax.axis_index("s")
    w = c * _NSC + s
    base = s * _NROWS

    _fill_rows(ones_v, _IW, 0.0)
    _zero_fill(acc, ones_v, base, semz)
    _fill_rows(ones_v, _IW, 1.0)
    plsc.subcore_barrier()

    def idx_rows(u):
        return dst_hbm.at[pl.ds(w * _RW + u * _SUP, _SUP)]

    def drain_sc():
        pltpu.make_async_copy(ones_v, acc.at[idxd_v.at[0, 0]], semsc).wait()

    pltpu.sync_copy(idx_rows(0), idxd_v.at[0])

    @pl.loop(0, _NQ)
    def _(q):
        pb = lax.rem(q, 2)

        @pl.when(q >= 1)
        def _():
            pltpu.make_async_copy(idx_rows(0), idxd_v.at[0], semi).wait()

        for r in range(_NB):
            if r < 2:
                @pl.when(q >= 1)
                def _():
                    for _j in range(_KJ):
                        drain_sc()
            else:
                for _j in range(_KJ):
                    drain_sc()
            for j in range(_KJ):
                pltpu.async_copy(
                    ones_v, acc.at[idxd_v.at[pb, _KJ * r + j]], semsc,
                    add=True)

        @pl.when(q + 1 < _NQ)
        def _():
            pltpu.async_copy(idx_rows(q + 1), idxd_v.at[1 - pb], semi)

    for _m in range(2):
        for _j in range(_KJ):
            drain_sc()

    plsc.subcore_barrier()
    pltpu.sync_copy(acc.at[pl.ds(base, _NROWS)],
                    out_hbm.at[c, pl.ds(base, _NROWS)])


@functools.cache
def _deg_sc_call():
    return pl.kernel(
        _deg_body,
        out_type=jax.ShapeDtypeStruct((_NC, _NP, 16), jnp.float32),
        mesh=_sc_mesh(),
        compiler_params=pltpu.CompilerParams(use_tc_tiling_on_sc=False),
        scratch_types=[
            pltpu.VMEM_SHARED((_NP, 16), jnp.float32),
            pltpu.VMEM((2, _SUP, _IW), jnp.int32),
            pltpu.VMEM((_IW, 16), jnp.float32),
            pltpu.SemaphoreType.DMA,
            pltpu.SemaphoreType.DMA,
            pltpu.SemaphoreType.DMA,
        ],
    )


def _edge_body(g_hbm, src_hbm, dst_hbm, out_hbm, acc, idxs_v, idxd_v, rows_v,
               semg0, semg1, semg2, semg3, sems0, sems1, sems2, sems3,
               semi, semz):
    c = lax.axis_index("c")
    s = lax.axis_index("s")
    w = c * _NSC + s
    base = s * _NROWS
    semg = (semg0, semg1, semg2, semg3)
    sems = (sems0, sems1, sems2, sems3)

    @pl.loop(0, _IW)
    def _(i):
        rows_v[0, 0, i] = jnp.zeros((16,), jnp.float32)

    _zero_fill(acc, rows_v.at[0, 0], base, semz)
    plsc.subcore_barrier()

    def sidx_rows(u):
        return src_hbm.at[pl.ds(w * _RW + u * _SUP, _SUP)]

    def didx_rows(u):
        return dst_hbm.at[pl.ds(w * _RW + u * _SUP, _SUP)]

    def fire_gathers(b, ib, r):
        for j in range(_KJ):
            pltpu.async_copy(
                g_hbm.at[idxs_v.at[ib, _KJ * r + j]], rows_v.at[b, j],
                semg[b])

    def drain_gathers(b):
        for j in range(_KJ):
            pltpu.make_async_copy(
                g_hbm.at[idxs_v.at[0, 0]], rows_v.at[b, j], semg[b]).wait()

    def fire_scatters(b, ib, r):
        for j in range(_KJ):
            pltpu.async_copy(
                rows_v.at[b, j], acc.at[idxd_v.at[ib, _KJ * r + j]], sems[b],
                add=True)

    def drain_scatters(b):
        for j in range(_KJ):
            pltpu.make_async_copy(
                rows_v.at[b, j], acc.at[idxd_v.at[0, 0]], sems[b]).wait()

    pltpu.sync_copy(sidx_rows(0), idxs_v.at[0])
    pltpu.sync_copy(didx_rows(0), idxd_v.at[0])

    @pl.loop(0, _NQ)
    def _(q):
        pb = lax.rem(q, 3)
        pbn = lax.rem(q + 2, 3)
        nxt = lax.rem(q + 1, 3)

        @pl.when(q >= 1)
        def _():
            pltpu.make_async_copy(sidx_rows(0), idxs_v.at[0], semi).wait()
            pltpu.make_async_copy(didx_rows(0), idxd_v.at[0], semi).wait()

        for r in range(_NB):
            b = r
            b2 = (r + 2) % _NB

            @pl.when(q >= 1)
            def _():
                drain_scatters(b)

            fire_gathers(b, pb, r)

            def _mid():
                drain_gathers(b2)
                if r < 2:
                    fire_scatters(b2, pbn, r + 2)
                else:
                    fire_scatters(b2, pb, r - 2)

            if r < 2:
                @pl.when(q >= 1)
                def _():
                    _mid()
            else:
                _mid()

        @pl.when(q + 1 < _NQ)
        def _():
            pltpu.async_copy(sidx_rows(q + 1), idxs_v.at[nxt], semi)
            pltpu.async_copy(didx_rows(q + 1), idxd_v.at[nxt], semi)

    pbl = (_NQ - 1) % 3
    for r in (2, 3):
        drain_gathers(r)
        fire_scatters(r, pbl, r)
    for b in range(_NB):
        drain_scatters(b)

    plsc.subcore_barrier()
    pltpu.sync_copy(acc.at[pl.ds(base, _NROWS)],
                    out_hbm.at[c, pl.ds(base, _NROWS)])


@functools.cache
def _edge_sc_call():
    return pl.kernel(
        _edge_body,
        out_type=jax.ShapeDtypeStruct((_NC, _NP, 16), jnp.float32),
        mesh=_sc_mesh(),
        compiler_params=pltpu.CompilerParams(use_tc_tiling_on_sc=False),
        scratch_types=[
            pltpu.VMEM_SHARED((_NP, 16), jnp.float32),
            pltpu.VMEM((3, _SUP, _IW), jnp.int32),
            pltpu.VMEM((3, _SUP, _IW), jnp.int32),
            pltpu.VMEM((_NB, _KJ, _IW, 16), jnp.float32),
            pltpu.SemaphoreType.DMA,
            pltpu.SemaphoreType.DMA,
            pltpu.SemaphoreType.DMA,
            pltpu.SemaphoreType.DMA,
            pltpu.SemaphoreType.DMA,
            pltpu.SemaphoreType.DMA,
            pltpu.SemaphoreType.DMA,
            pltpu.SemaphoreType.DMA,
            pltpu.SemaphoreType.DMA,
            pltpu.SemaphoreType.DMA,
        ],
    )


def _prep_body(x_ref, w_ref, dp_ref0, dp_ref1, zp_ref, dinv_ref):
    dv = lax.rsqrt(1.0 + dp_ref0[0] + dp_ref1[0])
    z = jnp.dot(x_ref[...], w_ref[...], preferred_element_type=jnp.float32)
    zp_ref[...] = z * dv
    dinv_ref[...] = dv


_prep_tc = pl.pallas_call(
    _prep_body,
    grid=(_GRID,),
    in_specs=[
        pl.BlockSpec((_BT, 128), lambda i: (i, 0)),
        pl.BlockSpec((128, 16), lambda i: (0, 0)),
        pl.BlockSpec((1, _BT, 16), lambda i: (0, i, 0)),
        pl.BlockSpec((1, _BT, 16), lambda i: (1, i, 0)),
    ],
    out_specs=[
        pl.BlockSpec((_BT, 16), lambda i: (i, 0)),
        pl.BlockSpec((_BT, 16), lambda i: (i, 0)),
    ],
    out_shape=[
        jax.ShapeDtypeStruct((_N, 16), jnp.float32),
        jax.ShapeDtypeStruct((_N, 16), jnp.float32),
    ],
)


def _layer_body(p_ref0, p_ref1, zp_ref, dinv_ref, w_ref, b_ref, out_ref):
    dv = dinv_ref[...]
    h = jnp.maximum(dv * (p_ref0[0] + p_ref1[0] + zp_ref[...])
                    + b_ref[...], 0.0)
    out_ref[...] = jnp.dot(
        h, w_ref[...], preferred_element_type=jnp.float32) * dv


_layer_tc = pl.pallas_call(
    _layer_body,
    grid=(_GRID,),
    in_specs=[
        pl.BlockSpec((1, _BT, 16), lambda i: (0, i, 0)),
        pl.BlockSpec((1, _BT, 16), lambda i: (1, i, 0)),
        pl.BlockSpec((_BT, 16), lambda i: (i, 0)),
        pl.BlockSpec((_BT, 16), lambda i: (i, 0)),
        pl.BlockSpec((16, 16), lambda i: (0, 0)),
        pl.BlockSpec((1, 16), lambda i: (0, 0)),
    ],
    out_specs=pl.BlockSpec((_BT, 16), lambda i: (i, 0)),
    out_shape=jax.ShapeDtypeStruct((_N, 16), jnp.float32),
)


def _final_body(p_ref0, p_ref1, zp_ref, dinv_ref, b_ref, bat_ref,
                wf1_ref, bf1_ref, wf2_ref, bf2_ref, out_ref, pool_ref):
    i = pl.program_id(0)
    h = jnp.maximum(dinv_ref[...] * (p_ref0[0] + p_ref1[0] + zp_ref[...])
                    + b_ref[...], 0.0)
    bat = bat_ref[0]
    gids = lax.broadcasted_iota(jnp.int32, (1, _G), 1)
    onehot = (bat == gids).astype(jnp.float32)
    part = lax.dot_general(
        onehot, h, (((0,), (0,)), ((), ())),
        preferred_element_type=jnp.float32)

    @pl.when(i == 0)
    def _():
        pool_ref[...] = jnp.zeros_like(pool_ref)

    pool_ref[...] += part

    @pl.when(i == _GRID - 1)
    def _():
        h2 = jnp.maximum(
            jnp.dot(pool_ref[...], wf1_ref[...],
                    preferred_element_type=jnp.float32) + bf1_ref[...], 0.0)
        out_ref[...] = jnp.dot(
            h2, wf2_ref[...], preferred_element_type=jnp.float32) + bf2_ref[...]


_final_tc = pl.pallas_call(
    _final_body,
    grid=(_GRID,),
    in_specs=[
        pl.BlockSpec((1, _BT, 16), lambda i: (0, i, 0)),
        pl.BlockSpec((1, _BT, 16), lambda i: (1, i, 0)),
        pl.BlockSpec((_BT, 16), lambda i: (i, 0)),
        pl.BlockSpec((_BT, 16), lambda i: (i, 0)),
        pl.BlockSpec((1, 16), lambda i: (0, 0)),
        pl.BlockSpec((1, _BT, 1), lambda i: (i, 0, 0)),
        pl.BlockSpec((16, 16), lambda i: (0, 0)),
        pl.BlockSpec((1, 16), lambda i: (0, 0)),
        pl.BlockSpec((16, 1), lambda i: (0, 0)),
        pl.BlockSpec((1, 1), lambda i: (0, 0)),
    ],
    out_specs=pl.BlockSpec((_G, 1), lambda i: (0, 0)),
    out_shape=jax.ShapeDtypeStruct((_G, 1), jnp.float32),
    scratch_shapes=[pltpu.VMEM((_G, 16), jnp.float32)],
)


def kernel(x, edge_index, batch, W0, b0, W1, b1, W2, b2, W3, b3, W4, b4,
           W_fc1, b_fc1, W_fc2, b_fc2):
    pad = _EP - _E
    src = jnp.concatenate(
        [edge_index[0], jnp.zeros((pad,), jnp.int32)]).reshape(_R, _IW)
    dst = jnp.concatenate(
        [edge_index[1], jnp.full((pad,), _N, jnp.int32)]).reshape(_R, _IW)

    degp = _deg_sc_call()(dst)
    zp, dinv = _prep_tc(x, W0, degp, degp)

    for W, b_prev in ((W1, b0), (W2, b1), (W3, b2), (W4, b3)):
        p = _edge_sc_call()(zp, src, dst)
        zp = _layer_tc(p, p, zp, dinv, W, b_prev.reshape(1, 16))

    p = _edge_sc_call()(zp, src, dst)
    return _final_tc(p, p, zp, dinv, b4.reshape(1, 16),
                     batch.reshape(_GRID, _BT, 1),
                     W_fc1, b_fc1.reshape(1, 16),
                     W_fc2, b_fc2.reshape(1, 1))


# speedup vs baseline: 1.0437x; 1.0437x over previous
import functools

import jax
import jax.numpy as jnp
from jax import lax
from jax.experimental import pallas as pl
from jax.experimental.pallas import tpu as pltpu
from jax.experimental.pallas import tpu_sc as plsc

_N = 100000
_E = 3200000
_G = 64

_NC = 2
_NSC = 16
_NW = _NC * _NSC

_IW = 128
_KJ = 2
_EP = 3276800
_R = _EP // _IW
_RW = _R // _NW
_NM = _RW // _KJ
_NB = 4
_SUP = _NB * _KJ
_NQ = _NM // _NB

_NP = 100352
_NROWS = _NP // _NSC
_NZ = _NROWS // _IW

_BT = 10000
_GRID = _N // _BT


@functools.cache
def _sc_mesh():
    return plsc.VectorSubcoreMesh(
        core_axis_name="c", subcore_axis_name="s",
        num_cores=_NC, num_subcores=_NSC)


def _fill_rows(buf, n, val):
    @pl.loop(0, n)
    def _(i):
        buf[i] = jnp.full((16,), val, jnp.float32)


def _zero_fill(acc, zbuf, base, semz):
    @pl.loop(0, _NZ)
    def _(k):
        pltpu.async_copy(zbuf, acc.at[pl.ds(base + k * _IW, _IW)], semz)

    @pl.loop(0, _NZ)
    def _(k):
        pltpu.make_async_copy(
            zbuf, acc.at[pl.ds(base, _IW)], semz).wait()


def _deg_body(dst_hbm, out_hbm, acc, idxd_v, ones_v, semi, semsc, semz):
    c = lax.axis_index("c")
    s = lax.axis_index("s")
    w = c * _NSC + s
    base = s * _NROWS

    _fill_rows(ones_v, _IW, 0.0)
    _zero_fill(acc, ones_v, base, semz)
    _fill_rows(ones_v, _IW, 1.0)
    plsc.subcore_barrier()

    def idx_rows(u):
        return dst_hbm.at[pl.ds(w * _RW + u * _SUP, _SUP)]

    def drain_sc():
        pltpu.make_async_copy(ones_v, acc.at[idxd_v.at[0, 0]], semsc).wait()

    pltpu.sync_copy(idx_rows(0), idxd_v.at[0])

    @pl.loop(0, _NQ)
    def _(q):
        pb = lax.rem(q, 2)

        @pl.when(q >= 1)
        def _():
            pltpu.make_async_copy(idx_rows(0), idxd_v.at[0], semi).wait()

        for r in range(_NB):
            if r < 2:
                @pl.when(q >= 1)
                def _():
                    for _j in range(_KJ):
                        drain_sc()
            else:
                for _j in range(_KJ):
                    drain_sc()
            for j in range(_KJ):
                pltpu.async_copy(
                    ones_v, acc.at[idxd_v.at[pb, _KJ * r + j]], semsc,
                    add=True)

        @pl.when(q + 1 < _NQ)
        def _():
            pltpu.async_copy(idx_rows(q + 1), idxd_v.at[1 - pb], semi)

    for _m in range(2):
        for _j in range(_KJ):
            drain_sc()

    plsc.subcore_barrier()
    pltpu.sync_copy(acc.at[pl.ds(base, _NROWS)],
                    out_hbm.at[c, pl.ds(base, _NROWS)])


@functools.cache
def _deg_sc_call():
    return pl.kernel(
        _deg_body,
        out_type=jax.ShapeDtypeStruct((_NC, _NP, 16), jnp.float32),
        mesh=_sc_mesh(),
        compiler_params=pltpu.CompilerParams(use_tc_tiling_on_sc=False),
        scratch_types=[
            pltpu.VMEM_SHARED((_NP, 16), jnp.float32),
            pltpu.VMEM((2, _SUP, _IW), jnp.int32),
            pltpu.VMEM((_IW, 16), jnp.float32),
            pltpu.SemaphoreType.DMA,
            pltpu.SemaphoreType.DMA,
            pltpu.SemaphoreType.DMA,
        ],
    )


def _edge_body(g_hbm, src_hbm, dst_hbm, out_hbm, acc, idxs_v, idxd_v, rows_v,
               semg0, semg1, semg2, semg3, sems0, sems1, sems2, sems3,
               semi, semz):
    c = lax.axis_index("c")
    s = lax.axis_index("s")
    w = c * _NSC + s
    base = s * _NROWS
    semg = (semg0, semg1, semg2, semg3)
    sems = (sems0, sems1, sems2, sems3)

    @pl.loop(0, _IW)
    def _(i):
        rows_v[0, 0, i] = jnp.zeros((16,), jnp.float32)

    _zero_fill(acc, rows_v.at[0, 0], base, semz)
    plsc.subcore_barrier()

    def sidx_rows(u):
        return src_hbm.at[pl.ds(w * _RW + u * _SUP, _SUP)]

    def didx_rows(u):
        return dst_hbm.at[pl.ds(w * _RW + u * _SUP, _SUP)]

    def fire_gathers(b, ib, r):
        for j in range(_KJ):
            pltpu.async_copy(
                g_hbm.at[idxs_v.at[ib, _KJ * r + j]], rows_v.at[b, j],
                semg[b])

    def drain_gathers(b):
        for j in range(_KJ):
            pltpu.make_async_copy(
                g_hbm.at[idxs_v.at[0, 0]], rows_v.at[b, j], semg[b]).wait()

    def fire_scatters(b, ib, r):
        for j in range(_KJ):
            pltpu.async_copy(
                rows_v.at[b, j], acc.at[idxd_v.at[ib, _KJ * r + j]], sems[b],
                add=True)

    def drain_scatters(b):
        for j in range(_KJ):
            pltpu.make_async_copy(
                rows_v.at[b, j], acc.at[idxd_v.at[0, 0]], sems[b]).wait()

    pltpu.sync_copy(sidx_rows(0), idxs_v.at[0])
    pltpu.sync_copy(didx_rows(0), idxd_v.at[0])

    @pl.loop(0, _NQ)
    def _(q):
        pb = lax.rem(q, 3)
        pbn = lax.rem(q + 2, 3)
        nxt = lax.rem(q + 1, 3)

        @pl.when(q >= 1)
        def _():
            pltpu.make_async_copy(sidx_rows(0), idxs_v.at[0], semi).wait()
            pltpu.make_async_copy(didx_rows(0), idxd_v.at[0], semi).wait()

        for r in range(_NB):
            b = r
            b2 = (r + 2) % _NB

            @pl.when(q >= 1)
            def _():
                drain_scatters(b)

            fire_gathers(b, pb, r)

            def _mid():
                drain_gathers(b2)
                if r < 2:
                    fire_scatters(b2, pbn, r + 2)
                else:
                    fire_scatters(b2, pb, r - 2)

            if r < 2:
                @pl.when(q >= 1)
                def _():
                    _mid()
            else:
                _mid()

        @pl.when(q + 1 < _NQ)
        def _():
            pltpu.async_copy(sidx_rows(q + 1), idxs_v.at[nxt], semi)
            pltpu.async_copy(didx_rows(q + 1), idxd_v.at[nxt], semi)

    pbl = (_NQ - 1) % 3
    for r in (2, 3):
        drain_gathers(r)
        fire_scatters(r, pbl, r)
    for b in range(_NB):
        drain_scatters(b)

    plsc.subcore_barrier()
    pltpu.sync_copy(acc.at[pl.ds(base, _NROWS)],
                    out_hbm.at[c, pl.ds(base, _NROWS)])


@functools.cache
def _edge_sc_call():
    return pl.kernel(
        _edge_body,
        out_type=jax.ShapeDtypeStruct((_NC, _NP, 16), jnp.float32),
        mesh=_sc_mesh(),
        compiler_params=pltpu.CompilerParams(use_tc_tiling_on_sc=False),
        scratch_types=[
            pltpu.VMEM_SHARED((_NP, 16), jnp.float32),
            pltpu.VMEM((3, _SUP, _IW), jnp.int32),
            pltpu.VMEM((3, _SUP, _IW), jnp.int32),
            pltpu.VMEM((_NB, _KJ, _IW, 16), jnp.float32),
            pltpu.SemaphoreType.DMA,
            pltpu.SemaphoreType.DMA,
            pltpu.SemaphoreType.DMA,
            pltpu.SemaphoreType.DMA,
            pltpu.SemaphoreType.DMA,
            pltpu.SemaphoreType.DMA,
            pltpu.SemaphoreType.DMA,
            pltpu.SemaphoreType.DMA,
            pltpu.SemaphoreType.DMA,
            pltpu.SemaphoreType.DMA,
        ],
    )


def _prep_body(x_ref, w_ref, dp_ref0, dp_ref1, zp_ref, dinv_ref):
    dv = lax.rsqrt(1.0 + dp_ref0[0] + dp_ref1[0])
    z = jnp.dot(x_ref[...], w_ref[...], preferred_element_type=jnp.float32)
    zp_ref[...] = z * dv
    dinv_ref[...] = dv


_prep_tc = pl.pallas_call(
    _prep_body,
    grid=(_GRID,),
    in_specs=[
        pl.BlockSpec((_BT, 128), lambda i: (i, 0)),
        pl.BlockSpec((128, 16), lambda i: (0, 0)),
        pl.BlockSpec((1, _BT, 16), lambda i: (0, i, 0)),
        pl.BlockSpec((1, _BT, 16), lambda i: (1, i, 0)),
    ],
    out_specs=[
        pl.BlockSpec((_BT, 16), lambda i: (i, 0)),
        pl.BlockSpec((_BT, 16), lambda i: (i, 0)),
    ],
    out_shape=[
        jax.ShapeDtypeStruct((_N, 16), jnp.float32),
        jax.ShapeDtypeStruct((_N, 16), jnp.float32),
    ],
)


def _layer_body(p_ref0, p_ref1, zp_ref, dinv_ref, w_ref, b_ref, out_ref):
    dv = dinv_ref[...]
    h = jnp.maximum(dv * (p_ref0[0] + p_ref1[0] + zp_ref[...])
                    + b_ref[...], 0.0)
    out_ref[...] = jnp.dot(
        h, w_ref[...], preferred_element_type=jnp.float32) * dv


_layer_tc = pl.pallas_call(
    _layer_body,
    grid=(_GRID,),
    in_specs=[
        pl.BlockSpec((1, _BT, 16), lambda i: (0, i, 0)),
        pl.BlockSpec((1, _BT, 16), lambda i: (1, i, 0)),
        pl.BlockSpec((_BT, 16), lambda i: (i, 0)),
        pl.BlockSpec((_BT, 16), lambda i: (i, 0)),
        pl.BlockSpec((16, 16), lambda i: (0, 0)),
        pl.BlockSpec((1, 16), lambda i: (0, 0)),
    ],
    out_specs=pl.BlockSpec((_BT, 16), lambda i: (i, 0)),
    out_shape=jax.ShapeDtypeStruct((_N, 16), jnp.float32),
)


def _final_body(p_ref0, p_ref1, zp_ref, dinv_ref, b_ref, bat_ref,
                wf1_ref, bf1_ref, wf2_ref, bf2_ref, out_ref, pool_ref):
    i = pl.program_id(0)
    h = jnp.maximum(dinv_ref[...] * (p_ref0[0] + p_ref1[0] + zp_ref[...])
                    + b_ref[...], 0.0)
    bat = bat_ref[0]
    gids = lax.broadcasted_iota(jnp.int32, (1, _G), 1)
    onehot = (bat == gids).astype(jnp.float32)
    part = lax.dot_general(
        onehot, h, (((0,), (0,)), ((), ())),
        preferred_element_type=jnp.float32)

    @pl.when(i == 0)
    def _():
        pool_ref[...] = jnp.zeros_like(pool_ref)

    pool_ref[...] += part

    @pl.when(i == _GRID - 1)
    def _():
        h2 = jnp.maximum(
            jnp.dot(pool_ref[...], wf1_ref[...],
                    preferred_element_type=jnp.float32) + bf1_ref[...], 0.0)
        out_ref[...] = jnp.dot(
            h2, wf2_ref[...], preferred_element_type=jnp.float32) + bf2_ref[...]


_final_tc = pl.pallas_call(
    _final_body,
    grid=(_GRID,),
    in_specs=[
        pl.BlockSpec((1, _BT, 16), lambda i: (0, i, 0)),
        pl.BlockSpec((1, _BT, 16), lambda i: (1, i, 0)),
        pl.BlockSpec((_BT, 16), lambda i: (i, 0)),
        pl.BlockSpec((_BT, 16), lambda i: (i, 0)),
        pl.BlockSpec((1, 16), lambda i: (0, 0)),
        pl.BlockSpec((1, _BT, 1), lambda i: (i, 0, 0)),
        pl.BlockSpec((16, 16), lambda i: (0, 0)),
        pl.BlockSpec((1, 16), lambda i: (0, 0)),
        pl.BlockSpec((16, 1), lambda i: (0, 0)),
        pl.BlockSpec((1, 1), lambda i: (0, 0)),
    ],
    out_specs=pl.BlockSpec((_G, 1), lambda i: (0, 0)),
    out_shape=jax.ShapeDtypeStruct((_G, 1), jnp.float32),
    scratch_shapes=[pltpu.VMEM((_G, 16), jnp.float32)],
)


def kernel(x, edge_index, batch, W0, b0, W1, b1, W2, b2, W3, b3, W4, b4,
           W_fc1, b_fc1, W_fc2, b_fc2):
    pad = _EP - _E
    src = jnp.concatenate(
        [edge_index[0], jnp.zeros((pad,), jnp.int32)]).reshape(_R, _IW)
    dst = jnp.concatenate(
        [edge_index[1], jnp.full((pad,), _N, jnp.int32)]).reshape(_R, _IW)

    degp = _deg_sc_call()(dst)
    zp, dinv = _prep_tc(x, W0, degp, degp)

    for W, b_prev in ((W1, b0), (W2, b1), (W3, b2), (W4, b3)):
        p = _edge_sc_call()(zp, src, dst)
        zp = _layer_tc(p, p, zp, dinv, W, b_prev.reshape(1, 16))

    p = _edge_sc_call()(zp, src, dst)
    return _final_tc(p, p, zp, dinv, b4.reshape(1, 16),
                     batch.reshape(_GRID, _BT, 1),
                     W_fc1, b_fc1.reshape(1, 16),
                     W_fc2, b_fc2.reshape(1, 1))
